# Initial kernel scaffold; baseline (speedup 1.0000x reference)
import jax
import jax.numpy as jnp
from jax.experimental import pallas as pl


def kernel(x, edge_index, edge_probs, W1, b1, W2, b2, W3, b3, Wm1, bm1, Wm2, bm2):
    raise NotImplementedError("write your pallas kernel here")


# speedup vs baseline: 10.7695x; 10.7695x over previous
import functools

import jax
import jax.numpy as jnp
from jax import lax
from jax.experimental import pallas as pl
from jax.experimental.pallas import tpu as pltpu
from jax.experimental.pallas import tpu_sc as plsc

_NC = 2
_NS = 16
_NW = _NC * _NS
_BLK = 128


def _mesh():
    return plsc.VectorSubcoreMesh(core_axis_name="c", subcore_axis_name="s")


def _sc_deg(dst, ew, zeros_n):
    n = zeros_n.shape[0]
    e = dst.shape[0]
    epc, epw = e // _NC, e // _NW
    nblk = epw // _BLK
    rps = n // _NS

    def body(dst_hbm, ew_hbm, z_hbm, out_hbm, di, ev, acc):
        c = lax.axis_index("c")
        s = lax.axis_index("s")
        pltpu.sync_copy(z_hbm.at[pl.ds(s * rps, rps)], acc.at[pl.ds(s * rps, rps)])
        plsc.subcore_barrier()
        base0 = c * epc + s * epw

        def blk(i, carry):
            base = base0 + i * _BLK
            pltpu.sync_copy(dst_hbm.at[pl.ds(base, _BLK)], di)
            pltpu.sync_copy(ew_hbm.at[pl.ds(base, _BLK)], ev)
            pltpu.sync_copy(ev, acc.at[di], add=True)
            return carry

        lax.fori_loop(0, nblk, blk, 0)
        plsc.subcore_barrier()
        pltpu.sync_copy(acc.at[pl.ds(s * rps, rps)],
                        out_hbm.at[c, pl.ds(s * rps, rps)])

    kfn = pl.kernel(
        body,
        out_type=jax.ShapeDtypeStruct((_NC, n), jnp.float32),
        mesh=_mesh(),
        compiler_params=pltpu.CompilerParams(needs_layout_passes=False, use_tc_tiling_on_sc=False),
        scratch_types=[
            pltpu.VMEM((_BLK,), jnp.int32),
            pltpu.VMEM((_BLK,), jnp.float32),
            pltpu.VMEM_SHARED((n,), jnp.float32),
        ],
    )
    return kfn(dst, ew, zeros_n)


def _sc_norm(src, dst, ew, dis):
    e = src.shape[0]
    n = dis.shape[0]
    epc, epw = e // _NC, e // _NW
    nblk = epw // _BLK

    def body(src_hbm, dst_hbm, ew_hbm, dis_hbm, out_hbm, si, di, ev, ov, dv):
        c = lax.axis_index("c")
        s = lax.axis_index("s")
        pltpu.sync_copy(dis_hbm, dv)
        base0 = c * epc + s * epw

        def blk(i, carry):
            base = base0 + i * _BLK
            pltpu.sync_copy(src_hbm.at[pl.ds(base, _BLK)], si)
            pltpu.sync_copy(dst_hbm.at[pl.ds(base, _BLK)], di)
            pltpu.sync_copy(ew_hbm.at[pl.ds(base, _BLK)], ev)
            for j in range(_BLK // 16):
                sl = pl.ds(j * 16, 16)
                a = plsc.load_gather(dv, [si[sl]])
                b = plsc.load_gather(dv, [di[sl]])
                ov[sl] = a * ev[sl] * b
            pltpu.sync_copy(ov, out_hbm.at[pl.ds(base, _BLK)])
            return carry

        lax.fori_loop(0, nblk, blk, 0)

    kfn = pl.kernel(
        body,
        out_type=jax.ShapeDtypeStruct((e,), jnp.float32),
        mesh=_mesh(),
        compiler_params=pltpu.CompilerParams(needs_layout_passes=False, use_tc_tiling_on_sc=False),
        scratch_types=[
            pltpu.VMEM((_BLK,), jnp.int32),
            pltpu.VMEM((_BLK,), jnp.int32),
            pltpu.VMEM((_BLK,), jnp.float32),
            pltpu.VMEM((_BLK,), jnp.float32),
            pltpu.VMEM((n,), jnp.float32),
        ],
    )
    return kfn(src, dst, ew, dis)


def _sc_spmm(u, src, dst, norm, zeros_nf):
    n, f = u.shape
    e = src.shape[0]
    epc, epw = e // _NC, e // _NW
    nblk = epw // _BLK

    def body(u_hbm, src_hbm, dst_hbm, norm_hbm, z_hbm, out_hbm,
             si, di, nv, rows, scaled, acc):
        c = lax.axis_index("c")
        s = lax.axis_index("s")
        rps = n // _NS
        pltpu.sync_copy(z_hbm.at[pl.ds(s * rps, rps)],
                        acc.at[pl.ds(s * rps, rps)])
        plsc.subcore_barrier()
        base0 = c * epc + s * epw

        def blk(i, carry):
            base = base0 + i * _BLK
            pltpu.sync_copy(src_hbm.at[pl.ds(base, _BLK)], si)
            pltpu.sync_copy(dst_hbm.at[pl.ds(base, _BLK)], di)
            pltpu.sync_copy(norm_hbm.at[pl.ds(base, _BLK)], nv)
            pltpu.sync_copy(u_hbm.at[si], rows)
            for j in range(_BLK):
                nb = plsc.load_gather(nv, [jnp.full((16,), j, jnp.int32)])
                scaled[j] = rows[j] * nb
            pltpu.sync_copy(scaled, acc.at[di], add=True)
            return carry

        lax.fori_loop(0, nblk, blk, 0)
        plsc.subcore_barrier()
        pltpu.sync_copy(acc.at[pl.ds(s * rps, rps)],
                        out_hbm.at[c, pl.ds(s * rps, rps)])

    kfn = pl.kernel(
        body,
        out_type=jax.ShapeDtypeStruct((_NC, n, f), jnp.float32),
        mesh=_mesh(),
        compiler_params=pltpu.CompilerParams(needs_layout_passes=False, use_tc_tiling_on_sc=False),
        scratch_types=[
            pltpu.VMEM((_BLK,), jnp.int32),
            pltpu.VMEM((_BLK,), jnp.int32),
            pltpu.VMEM((_BLK,), jnp.float32),
            pltpu.VMEM((_BLK, f), jnp.float32),
            pltpu.VMEM((_BLK, f), jnp.float32),
            pltpu.VMEM_SHARED((n, f), jnp.float32),
        ],
    )
    return kfn(u, src, dst, norm, zeros_nf)


def _tc_mm(x, w, npad):
    n, f = x.shape[0], w.shape[1]

    def body(x_ref, w_ref, o_ref):
        o_ref[pl.ds(0, n), :] = jnp.dot(x_ref[...], w_ref[...],
                                        preferred_element_type=jnp.float32)
        o_ref[pl.ds(n, npad - n), :] = jnp.zeros((npad - n, f), jnp.float32)

    return pl.pallas_call(
        body, out_shape=jax.ShapeDtypeStruct((npad, f), jnp.float32))(x, w)


def _tc_dis(degp, n):
    npad = degp.shape[1]

    def body(p_ref, o_ref):
        deg = (p_ref[0, pl.ds(0, n)] + p_ref[1, pl.ds(0, n)]).reshape(n, 1)
        safe = jnp.where(deg > 0, deg, 1.0)
        o_ref[pl.ds(0, n), :] = jnp.where(deg > 0, lax.rsqrt(safe), 0.0)
        o_ref[pl.ds(n, npad - n), :] = jnp.zeros((npad - n, 1), jnp.float32)

    return pl.pallas_call(
        body, out_shape=jax.ShapeDtypeStruct((npad, 1), jnp.float32))(degp)


def _tc_combine_mm(p, b, w, n):
    npad, f = p.shape[1], w.shape[1]

    def body(p_ref, b_ref, w_ref, o_ref):
        h = jnp.maximum(p_ref[0, pl.ds(0, n), :] + p_ref[1, pl.ds(0, n), :]
                        + b_ref[...], 0.0)
        o_ref[pl.ds(0, n), :] = jnp.dot(h, w_ref[...],
                                        preferred_element_type=jnp.float32)
        o_ref[pl.ds(n, npad - n), :] = jnp.zeros((npad - n, f), jnp.float32)

    return pl.pallas_call(
        body, out_shape=jax.ShapeDtypeStruct((npad, f), jnp.float32))(
            p, b.reshape(1, -1), w)


def _tc_head(p, b3, wm1, bm1, wm2, bm2, n):
    def body(p_ref, b3_ref, wm1_ref, bm1_ref, wm2_ref, bm2_ref, o_ref):
        h3 = jnp.maximum(p_ref[0, pl.ds(0, n), :] + p_ref[1, pl.ds(0, n), :]
                         + b3_ref[...], 0.0)
        h4 = jnp.maximum(
            jnp.dot(h3, wm1_ref[...], preferred_element_type=jnp.float32)
            + bm1_ref[...], 0.0)
        h5 = (jnp.dot(h4, wm2_ref[...], preferred_element_type=jnp.float32)
              + bm2_ref[...])
        mn = jnp.min(h5)
        mx = jnp.max(h5)
        o_ref[...] = (h5 - mn) / (mx - mn)

    return pl.pallas_call(
        body, out_shape=jax.ShapeDtypeStruct((n, 1), jnp.float32))(
            p, b3.reshape(1, -1), wm1, bm1.reshape(1, -1), wm2,
            bm2.reshape(1, 1))


def kernel(x, edge_index, edge_probs, W1, b1, W2, b2, W3, b3, Wm1, bm1, Wm2, bm2):
    n = x.shape[0]
    hid = W1.shape[1]
    out_ch = W3.shape[1]
    i32 = jnp.int32

    loop = jnp.arange(n, dtype=i32)
    src = jnp.concatenate([edge_index[0].astype(i32), loop])
    dst = jnp.concatenate([edge_index[1].astype(i32), loop])
    ew = jnp.concatenate([edge_probs.astype(jnp.float32),
                          jnp.ones((n,), jnp.float32)])
    e = src.shape[0]

    real = _BLK - 1
    nblocks = ((-(-e // real) + _NW - 1) // _NW) * _NW
    pad = nblocks * real - e
    src = jnp.pad(src, (0, pad))
    dst = jnp.pad(dst, (0, pad), constant_values=n)
    ew = jnp.pad(ew, (0, pad))
    src = jnp.concatenate(
        [jnp.zeros((nblocks, 1), i32), src.reshape(nblocks, real)], 1).ravel()
    dst = jnp.concatenate(
        [jnp.full((nblocks, 1), n, i32), dst.reshape(nblocks, real)], 1).ravel()
    ew = jnp.concatenate(
        [jnp.zeros((nblocks, 1), jnp.float32), ew.reshape(nblocks, real)],
        1).ravel()

    npad = ((n + _NS * 8) // (_NS * 8)) * (_NS * 8)
    zeros_n = jnp.zeros((npad,), jnp.float32)
    zeros_nf = jnp.zeros((npad, hid), jnp.float32)

    degp = _sc_deg(dst, ew, zeros_n)
    dis = _tc_dis(degp, n)
    norm = _sc_norm(src, dst, ew, dis.reshape(npad))

    u1 = _tc_mm(x, W1, npad)
    p1 = _sc_spmm(u1, src, dst, norm, zeros_nf)
    u2 = _tc_combine_mm(p1, b1, W2, n)
    p2 = _sc_spmm(u2, src, dst, norm, zeros_nf)
    w3p = jnp.concatenate(
        [W3, jnp.zeros((hid, hid - out_ch), jnp.float32)], axis=1)
    u3 = _tc_combine_mm(p2, b2, w3p, n)
    p3 = _sc_spmm(u3, src, dst, norm, zeros_nf)

    b3p = jnp.concatenate([b3, jnp.zeros((hid - out_ch,), jnp.float32)])
    wm1p = jnp.concatenate(
        [Wm1, jnp.zeros((hid - out_ch, Wm1.shape[1]), jnp.float32)], axis=0)
    return _tc_head(p3, b3p, wm1p, bm1, Wm2, bm2, n)


# speedup vs baseline: 17.0395x; 1.5822x over previous
import functools

import jax
import jax.numpy as jnp
from jax import lax
from jax.experimental import pallas as pl
from jax.experimental.pallas import tpu as pltpu
from jax.experimental.pallas import tpu_sc as plsc

_NC = 2
_NS = 16
_NW = _NC * _NS
_BLK = 128


def _mesh():
    return plsc.VectorSubcoreMesh(core_axis_name="c", subcore_axis_name="s")


def _sc_deg(dst, ew, zeros_n):
    n = zeros_n.shape[0]
    e = dst.shape[0]
    epc, epw = e // _NC, e // _NW
    nblk = epw // _BLK
    rps = n // _NS

    def body(dst_hbm, ew_hbm, z_hbm, out_hbm, di, ev, acc):
        c = lax.axis_index("c")
        s = lax.axis_index("s")
        pltpu.sync_copy(z_hbm.at[pl.ds(s * rps, rps)], acc.at[pl.ds(s * rps, rps)])
        plsc.subcore_barrier()
        base0 = c * epc + s * epw

        def blk(i, carry):
            base = base0 + i * _BLK
            pltpu.sync_copy(dst_hbm.at[pl.ds(base, _BLK)], di)
            pltpu.sync_copy(ew_hbm.at[pl.ds(base, _BLK)], ev)
            pltpu.sync_copy(ev, acc.at[di], add=True)
            return carry

        lax.fori_loop(0, nblk, blk, 0)
        plsc.subcore_barrier()
        pltpu.sync_copy(acc.at[pl.ds(s * rps, rps)],
                        out_hbm.at[c, pl.ds(s * rps, rps)])

    kfn = pl.kernel(
        body,
        out_type=jax.ShapeDtypeStruct((_NC, n), jnp.float32),
        mesh=_mesh(),
        compiler_params=pltpu.CompilerParams(needs_layout_passes=False, use_tc_tiling_on_sc=False),
        scratch_types=[
            pltpu.VMEM((_BLK,), jnp.int32),
            pltpu.VMEM((_BLK,), jnp.float32),
            pltpu.VMEM_SHARED((n,), jnp.float32),
        ],
    )
    return kfn(dst, ew, zeros_n)


def _sc_norm(src, dst, ew, dis):
    e = src.shape[0]
    n = dis.shape[0]
    epc, epw = e // _NC, e // _NW
    nblk = epw // _BLK

    def body(src_hbm, dst_hbm, ew_hbm, dis_hbm, out_hbm, si, di, ev, ov, dv):
        c = lax.axis_index("c")
        s = lax.axis_index("s")
        pltpu.sync_copy(dis_hbm, dv)
        base0 = c * epc + s * epw

        def blk(i, carry):
            base = base0 + i * _BLK
            pltpu.sync_copy(src_hbm.at[pl.ds(base, _BLK)], si)
            pltpu.sync_copy(dst_hbm.at[pl.ds(base, _BLK)], di)
            pltpu.sync_copy(ew_hbm.at[pl.ds(base, _BLK)], ev)
            for j in range(_BLK // 16):
                sl = pl.ds(j * 16, 16)
                a = plsc.load_gather(dv, [si[sl]])
                b = plsc.load_gather(dv, [di[sl]])
                ov[sl] = a * ev[sl] * b
            pltpu.sync_copy(ov, out_hbm.at[pl.ds(base, _BLK)])
            return carry

        lax.fori_loop(0, nblk, blk, 0)

    kfn = pl.kernel(
        body,
        out_type=jax.ShapeDtypeStruct((e,), jnp.float32),
        mesh=_mesh(),
        compiler_params=pltpu.CompilerParams(needs_layout_passes=False, use_tc_tiling_on_sc=False),
        scratch_types=[
            pltpu.VMEM((_BLK,), jnp.int32),
            pltpu.VMEM((_BLK,), jnp.int32),
            pltpu.VMEM((_BLK,), jnp.float32),
            pltpu.VMEM((_BLK,), jnp.float32),
            pltpu.VMEM((n,), jnp.float32),
        ],
    )
    return kfn(src, dst, ew, dis)


def _sc_spmm(u, srcnorm, dstb, zeros_nf):
    n, f = u.shape
    nblocks = srcnorm.shape[0]
    nblk = nblocks // _NW
    rps = n // _NS

    def body(u_hbm, sn_hbm, db_hbm, z_hbm, out_hbm,
             recA, recB, diA, diB, rowsA, rowsB, scA, scB,
             srA, sdA, sgA, srB, sdB, sgB, acc):
        c = lax.axis_index("c")
        s = lax.axis_index("s")
        pltpu.sync_copy(z_hbm.at[pl.ds(s * rps, rps)], acc.at[pl.ds(s * rps, rps)])
        plsc.subcore_barrier()
        base0 = (c * _NS + s) * nblk

        def start_rec(b, rec, di, sr, sd):
            pltpu.async_copy(sn_hbm.at[b], rec, sr)
            pltpu.async_copy(db_hbm.at[b], di, sd)

        def wait_rec(rec, di, sr, sd):
            pltpu.make_async_copy(sn_hbm.at[0], rec, sr).wait()
            pltpu.make_async_copy(db_hbm.at[0], di, sd).wait()

        def scale(rec, rows, sc):
            for j in range(_BLK):
                nb = plsc.load_gather(
                    rec, [jnp.full((16,), 1, jnp.int32),
                          jnp.full((16,), j, jnp.int32)])
                sc[j] = rows[j] * plsc.bitcast(nb, jnp.float32)

        start_rec(base0, recA, diA, srA, sdA)
        wait_rec(recA, diA, srA, sdA)
        pltpu.async_copy(u_hbm.at[recA.at[0]], rowsA, sgA)
        start_rec(base0 + 1, recB, diB, srB, sdB)

        def it(i, carry):
            a = base0 + 2 * i
            pltpu.make_async_copy(u_hbm.at[recA.at[0]], rowsA, sgA).wait()
            wait_rec(recB, diB, srB, sdB)
            pltpu.async_copy(u_hbm.at[recB.at[0]], rowsB, sgB)
            scale(recA, rowsA, scA)
            pltpu.sync_copy(scA, acc.at[diA], add=True)

            @pl.when(2 * i + 2 < nblk)
            def _():
                start_rec(a + 2, recA, diA, srA, sdA)

            pltpu.make_async_copy(u_hbm.at[recB.at[0]], rowsB, sgB).wait()
            scale(recB, rowsB, scB)
            pltpu.sync_copy(scB, acc.at[diB], add=True)

            @pl.when(2 * i + 3 < nblk)
            def _():
                start_rec(a + 3, recB, diB, srB, sdB)

            @pl.when(2 * i + 2 < nblk)
            def _():
                wait_rec(recA, diA, srA, sdA)
                pltpu.async_copy(u_hbm.at[recA.at[0]], rowsA, sgA)

            return carry

        lax.fori_loop(0, nblk // 2, it, 0)
        plsc.subcore_barrier()
        pltpu.sync_copy(acc.at[pl.ds(s * rps, rps)],
                        out_hbm.at[c, pl.ds(s * rps, rps)])

    kfn = pl.kernel(
        body,
        out_type=jax.ShapeDtypeStruct((_NC, n, f), jnp.float32),
        mesh=_mesh(),
        compiler_params=pltpu.CompilerParams(needs_layout_passes=False, use_tc_tiling_on_sc=False),
        scratch_types=[
            pltpu.VMEM((2, _BLK), jnp.int32),
            pltpu.VMEM((2, _BLK), jnp.int32),
            pltpu.VMEM((_BLK,), jnp.int32),
            pltpu.VMEM((_BLK,), jnp.int32),
            pltpu.VMEM((_BLK, f), jnp.float32),
            pltpu.VMEM((_BLK, f), jnp.float32),
            pltpu.VMEM((_BLK, f), jnp.float32),
            pltpu.VMEM((_BLK, f), jnp.float32),
            pltpu.SemaphoreType.DMA,
            pltpu.SemaphoreType.DMA,
            pltpu.SemaphoreType.DMA,
            pltpu.SemaphoreType.DMA,
            pltpu.SemaphoreType.DMA,
            pltpu.SemaphoreType.DMA,
            pltpu.VMEM_SHARED((n, f), jnp.float32),
        ],
    )
    return kfn(u, srcnorm, dstb, zeros_nf)


def _tc_mm(x, w, npad):
    n, f = x.shape[0], w.shape[1]

    def body(x_ref, w_ref, o_ref):
        o_ref[pl.ds(0, n), :] = jnp.dot(x_ref[...], w_ref[...],
                                        preferred_element_type=jnp.float32)
        o_ref[pl.ds(n, npad - n), :] = jnp.zeros((npad - n, f), jnp.float32)

    return pl.pallas_call(
        body, out_shape=jax.ShapeDtypeStruct((npad, f), jnp.float32))(x, w)


def _tc_dis(degp, n):
    npad = degp.shape[1]

    def body(p_ref, o_ref):
        deg = (p_ref[0, pl.ds(0, n)] + p_ref[1, pl.ds(0, n)]).reshape(n, 1)
        safe = jnp.where(deg > 0, deg, 1.0)
        o_ref[pl.ds(0, n), :] = jnp.where(deg > 0, lax.rsqrt(safe), 0.0)
        o_ref[pl.ds(n, npad - n), :] = jnp.zeros((npad - n, 1), jnp.float32)

    return pl.pallas_call(
        body, out_shape=jax.ShapeDtypeStruct((npad, 1), jnp.float32))(degp)


def _tc_combine_mm(p, b, w, n):
    npad, f = p.shape[1], w.shape[1]

    def body(p_ref, b_ref, w_ref, o_ref):
        h = jnp.maximum(p_ref[0, pl.ds(0, n), :] + p_ref[1, pl.ds(0, n), :]
                        + b_ref[...], 0.0)
        o_ref[pl.ds(0, n), :] = jnp.dot(h, w_ref[...],
                                        preferred_element_type=jnp.float32)
        o_ref[pl.ds(n, npad - n), :] = jnp.zeros((npad - n, f), jnp.float32)

    return pl.pallas_call(
        body, out_shape=jax.ShapeDtypeStruct((npad, f), jnp.float32))(
            p, b.reshape(1, -1), w)


def _tc_head(p, b3, wm1, bm1, wm2, bm2, n):
    def body(p_ref, b3_ref, wm1_ref, bm1_ref, wm2_ref, bm2_ref, o_ref):
        h3 = jnp.maximum(p_ref[0, pl.ds(0, n), :] + p_ref[1, pl.ds(0, n), :]
                         + b3_ref[...], 0.0)
        h4 = jnp.maximum(
            jnp.dot(h3, wm1_ref[...], preferred_element_type=jnp.float32)
            + bm1_ref[...], 0.0)
        h5 = (jnp.dot(h4, wm2_ref[...], preferred_element_type=jnp.float32)
              + bm2_ref[...])
        mn = jnp.min(h5)
        mx = jnp.max(h5)
        o_ref[...] = (h5 - mn) / (mx - mn)

    return pl.pallas_call(
        body, out_shape=jax.ShapeDtypeStruct((n, 1), jnp.float32))(
            p, b3.reshape(1, -1), wm1, bm1.reshape(1, -1), wm2,
            bm2.reshape(1, 1))


def kernel(x, edge_index, edge_probs, W1, b1, W2, b2, W3, b3, Wm1, bm1, Wm2, bm2):
    n = x.shape[0]
    hid = W1.shape[1]
    out_ch = W3.shape[1]
    i32 = jnp.int32

    loop = jnp.arange(n, dtype=i32)
    src = jnp.concatenate([edge_index[0].astype(i32), loop])
    dst = jnp.concatenate([edge_index[1].astype(i32), loop])
    ew = jnp.concatenate([edge_probs.astype(jnp.float32),
                          jnp.ones((n,), jnp.float32)])
    e = src.shape[0]

    real = _BLK - 1
    nblocks = ((-(-e // real) + 2 * _NW - 1) // (2 * _NW)) * (2 * _NW)
    pad = nblocks * real - e
    src = jnp.pad(src, (0, pad))
    dst = jnp.pad(dst, (0, pad), constant_values=n)
    ew = jnp.pad(ew, (0, pad))
    src = jnp.concatenate(
        [jnp.zeros((nblocks, 1), i32), src.reshape(nblocks, real)], 1).ravel()
    dst = jnp.concatenate(
        [jnp.full((nblocks, 1), n, i32), dst.reshape(nblocks, real)], 1).ravel()
    ew = jnp.concatenate(
        [jnp.zeros((nblocks, 1), jnp.float32), ew.reshape(nblocks, real)],
        1).ravel()

    npad = ((n + _NS * 8) // (_NS * 8)) * (_NS * 8)
    zeros_n = jnp.zeros((npad,), jnp.float32)
    zeros_nf = jnp.zeros((npad, hid), jnp.float32)

    degp = _sc_deg(dst, ew, zeros_n)
    dis = _tc_dis(degp, n)
    norm = _sc_norm(src, dst, ew, dis.reshape(npad))

    srcnorm = jnp.stack(
        [src.reshape(nblocks, _BLK),
         lax.bitcast_convert_type(norm, i32).reshape(nblocks, _BLK)], axis=1)
    dstb = dst.reshape(nblocks, _BLK)

    u1 = _tc_mm(x, W1, npad)
    p1 = _sc_spmm(u1, srcnorm, dstb, zeros_nf)
    u2 = _tc_combine_mm(p1, b1, W2, n)
    p2 = _sc_spmm(u2, srcnorm, dstb, zeros_nf)
    w3p = jnp.concatenate(
        [W3, jnp.zeros((hid, hid - out_ch), jnp.float32)], axis=1)
    u3 = _tc_combine_mm(p2, b2, w3p, n)
    p3 = _sc_spmm(u3, srcnorm, dstb, zeros_nf)

    b3p = jnp.concatenate([b3, jnp.zeros((hid - out_ch,), jnp.float32)])
    wm1p = jnp.concatenate(
        [Wm1, jnp.zeros((hid - out_ch, Wm1.shape[1]), jnp.float32)], axis=0)
    return _tc_head(p3, b3p, wm1p, bm1, Wm2, bm2, n)


# speedup vs baseline: 21.6617x; 1.2713x over previous
import functools

import jax
import jax.numpy as jnp
from jax import lax
from jax.experimental import pallas as pl
from jax.experimental.pallas import tpu as pltpu
from jax.experimental.pallas import tpu_sc as plsc

_NC = 2
_NS = 16
_NW = _NC * _NS
_BLK = 128


def _mesh():
    return plsc.VectorSubcoreMesh(core_axis_name="c", subcore_axis_name="s")


def _sc_deg(dstb, ewb, zeros_n):
    n = zeros_n.shape[0]
    nblocks = dstb.shape[0]
    nblk = nblocks // _NW
    rps = n // _NS

    def body(db_hbm, ew_hbm, z_hbm, out_hbm,
             diA, diB, evA, evB, sA, sB, acc):
        c = lax.axis_index("c")
        s = lax.axis_index("s")
        pltpu.sync_copy(z_hbm.at[pl.ds(s * rps, rps)], acc.at[pl.ds(s * rps, rps)])
        plsc.subcore_barrier()
        base0 = (c * _NS + s) * nblk

        def start(b, di, ev, sem):
            pltpu.async_copy(db_hbm.at[b], di, sem)
            pltpu.async_copy(ew_hbm.at[b], ev, sem)

        def wait(di, ev, sem):
            pltpu.make_async_copy(db_hbm.at[0], di, sem).wait()
            pltpu.make_async_copy(ew_hbm.at[0], ev, sem).wait()

        start(base0, diA, evA, sA)

        def it(i, carry):
            a = base0 + 2 * i

            @pl.when(2 * i + 1 < nblk)
            def _():
                start(a + 1, diB, evB, sB)

            wait(diA, evA, sA)
            pltpu.sync_copy(evA, acc.at[diA], add=True)

            @pl.when(2 * i + 2 < nblk)
            def _():
                start(a + 2, diA, evA, sA)

            @pl.when(2 * i + 1 < nblk)
            def _():
                wait(diB, evB, sB)
                pltpu.sync_copy(evB, acc.at[diB], add=True)

            return carry

        lax.fori_loop(0, (nblk + 1) // 2, it, 0)
        plsc.subcore_barrier()
        pltpu.sync_copy(acc.at[pl.ds(s * rps, rps)],
                        out_hbm.at[c, pl.ds(s * rps, rps)])

    kfn = pl.kernel(
        body,
        out_type=jax.ShapeDtypeStruct((_NC, n), jnp.float32),
        mesh=_mesh(),
        compiler_params=pltpu.CompilerParams(needs_layout_passes=False, use_tc_tiling_on_sc=False),
        scratch_types=[
            pltpu.VMEM((_BLK,), jnp.int32),
            pltpu.VMEM((_BLK,), jnp.int32),
            pltpu.VMEM((_BLK,), jnp.float32),
            pltpu.VMEM((_BLK,), jnp.float32),
            pltpu.SemaphoreType.DMA,
            pltpu.SemaphoreType.DMA,
            pltpu.VMEM_SHARED((n,), jnp.float32),
        ],
    )
    return kfn(dstb, ewb, zeros_n)


def _sc_norm(recb, dis):
    nblocks = recb.shape[0]
    n = dis.shape[0]
    nblk = nblocks // _NW

    def body(rec_hbm, dis_hbm, out_hbm, rA, rB, oA, oB, sA, sB, soA, soB, dv):
        c = lax.axis_index("c")
        s = lax.axis_index("s")
        pltpu.sync_copy(dis_hbm, dv)
        base0 = (c * _NS + s) * nblk

        def compute(r, o):
            for j in range(_BLK // 16):
                sl = pl.ds(j * 16, 16)
                a = plsc.load_gather(dv, [r[0, sl]])
                b = plsc.load_gather(dv, [r[1, sl]])
                o[sl] = a * plsc.bitcast(r[2, sl], jnp.float32) * b

        pltpu.async_copy(rec_hbm.at[base0], rA, sA)

        def it(i, carry):
            a = base0 + 2 * i

            @pl.when(2 * i + 1 < nblk)
            def _():
                pltpu.async_copy(rec_hbm.at[a + 1], rB, sB)

            pltpu.make_async_copy(rec_hbm.at[0], rA, sA).wait()

            @pl.when(i > 0)
            def _():
                pltpu.make_async_copy(oA, out_hbm.at[0], soA).wait()

            compute(rA, oA)
            pltpu.async_copy(oA, out_hbm.at[a], soA)

            @pl.when(2 * i + 2 < nblk)
            def _():
                pltpu.async_copy(rec_hbm.at[a + 2], rA, sA)

            @pl.when(2 * i + 1 < nblk)
            def _():
                pltpu.make_async_copy(rec_hbm.at[0], rB, sB).wait()

                @pl.when(i > 0)
                def _():
                    pltpu.make_async_copy(oB, out_hbm.at[0], soB).wait()

                compute(rB, oB)
                pltpu.async_copy(oB, out_hbm.at[a + 1], soB)

            return carry

        lax.fori_loop(0, (nblk + 1) // 2, it, 0)
        pltpu.make_async_copy(oA, out_hbm.at[0], soA).wait()
        pltpu.make_async_copy(oB, out_hbm.at[0], soB).wait()

    kfn = pl.kernel(
        body,
        out_type=jax.ShapeDtypeStruct((nblocks, _BLK), jnp.float32),
        mesh=_mesh(),
        compiler_params=pltpu.CompilerParams(needs_layout_passes=False, use_tc_tiling_on_sc=False),
        scratch_types=[
            pltpu.VMEM((3, _BLK), jnp.int32),
            pltpu.VMEM((3, _BLK), jnp.int32),
            pltpu.VMEM((_BLK,), jnp.float32),
            pltpu.VMEM((_BLK,), jnp.float32),
            pltpu.SemaphoreType.DMA,
            pltpu.SemaphoreType.DMA,
            pltpu.SemaphoreType.DMA,
            pltpu.SemaphoreType.DMA,
            pltpu.VMEM((n,), jnp.float32),
        ],
    )
    return kfn(recb, dis)


def _sc_spmm(u, srcnorm, dstb, zeros_nf):
    n, f = u.shape
    nblocks = srcnorm.shape[0]
    nblk = nblocks // _NW
    rps = n // _NS

    def body(u_hbm, sn_hbm, db_hbm, z_hbm, out_hbm,
             recA, recB, diA, diB, rowsA, rowsB, scA, scB,
             srA, sdA, sgA, srB, sdB, sgB, acc):
        c = lax.axis_index("c")
        s = lax.axis_index("s")
        pltpu.sync_copy(z_hbm.at[pl.ds(s * rps, rps)], acc.at[pl.ds(s * rps, rps)])
        plsc.subcore_barrier()
        base0 = (c * _NS + s) * nblk

        def start_rec(b, rec, di, sr, sd):
            pltpu.async_copy(sn_hbm.at[b], rec, sr)
            pltpu.async_copy(db_hbm.at[b], di, sd)

        def wait_rec(rec, di, sr, sd):
            pltpu.make_async_copy(sn_hbm.at[0], rec, sr).wait()
            pltpu.make_async_copy(db_hbm.at[0], di, sd).wait()

        def scale(rec, rows, sc):
            for j in range(_BLK):
                nb = plsc.load_gather(
                    rec, [jnp.full((16,), 1, jnp.int32),
                          jnp.full((16,), j, jnp.int32)])
                sc[j] = rows[j] * plsc.bitcast(nb, jnp.float32)

        start_rec(base0, recA, diA, srA, sdA)
        wait_rec(recA, diA, srA, sdA)
        pltpu.async_copy(u_hbm.at[recA.at[0]], rowsA, sgA)
        start_rec(base0 + 1, recB, diB, srB, sdB)

        def it(i, carry):
            a = base0 + 2 * i
            pltpu.make_async_copy(u_hbm.at[recA.at[0]], rowsA, sgA).wait()
            wait_rec(recB, diB, srB, sdB)
            pltpu.async_copy(u_hbm.at[recB.at[0]], rowsB, sgB)
            scale(recA, rowsA, scA)
            pltpu.sync_copy(scA, acc.at[diA], add=True)

            @pl.when(2 * i + 2 < nblk)
            def _():
                start_rec(a + 2, recA, diA, srA, sdA)

            pltpu.make_async_copy(u_hbm.at[recB.at[0]], rowsB, sgB).wait()
            scale(recB, rowsB, scB)
            pltpu.sync_copy(scB, acc.at[diB], add=True)

            @pl.when(2 * i + 3 < nblk)
            def _():
                start_rec(a + 3, recB, diB, srB, sdB)

            @pl.when(2 * i + 2 < nblk)
            def _():
                wait_rec(recA, diA, srA, sdA)
                pltpu.async_copy(u_hbm.at[recA.at[0]], rowsA, sgA)

            return carry

        lax.fori_loop(0, nblk // 2, it, 0)
        plsc.subcore_barrier()
        pltpu.sync_copy(acc.at[pl.ds(s * rps, rps)],
                        out_hbm.at[c, pl.ds(s * rps, rps)])

    kfn = pl.kernel(
        body,
        out_type=jax.ShapeDtypeStruct((_NC, n, f), jnp.float32),
        mesh=_mesh(),
        compiler_params=pltpu.CompilerParams(needs_layout_passes=False, use_tc_tiling_on_sc=False),
        scratch_types=[
            pltpu.VMEM((2, _BLK), jnp.int32),
            pltpu.VMEM((2, _BLK), jnp.int32),
            pltpu.VMEM((_BLK,), jnp.int32),
            pltpu.VMEM((_BLK,), jnp.int32),
            pltpu.VMEM((_BLK, f), jnp.float32),
            pltpu.VMEM((_BLK, f), jnp.float32),
            pltpu.VMEM((_BLK, f), jnp.float32),
            pltpu.VMEM((_BLK, f), jnp.float32),
            pltpu.SemaphoreType.DMA,
            pltpu.SemaphoreType.DMA,
            pltpu.SemaphoreType.DMA,
            pltpu.SemaphoreType.DMA,
            pltpu.SemaphoreType.DMA,
            pltpu.SemaphoreType.DMA,
            pltpu.VMEM_SHARED((n, f), jnp.float32),
        ],
    )
    return kfn(u, srcnorm, dstb, zeros_nf)


def _tc_mm(x, w, npad):
    n, f = x.shape[0], w.shape[1]

    def body(x_ref, w_ref, o_ref):
        o_ref[pl.ds(0, n), :] = jnp.dot(x_ref[...], w_ref[...],
                                        preferred_element_type=jnp.float32)
        o_ref[pl.ds(n, npad - n), :] = jnp.zeros((npad - n, f), jnp.float32)

    return pl.pallas_call(
        body, out_shape=jax.ShapeDtypeStruct((npad, f), jnp.float32))(x, w)


def _tc_dis(degp, n):
    npad = degp.shape[1]

    def body(p_ref, o_ref):
        deg = (p_ref[0, pl.ds(0, n)] + p_ref[1, pl.ds(0, n)]).reshape(n, 1)
        safe = jnp.where(deg > 0, deg, 1.0)
        o_ref[pl.ds(0, n), :] = jnp.where(deg > 0, lax.rsqrt(safe), 0.0)
        o_ref[pl.ds(n, npad - n), :] = jnp.zeros((npad - n, 1), jnp.float32)

    return pl.pallas_call(
        body, out_shape=jax.ShapeDtypeStruct((npad, 1), jnp.float32))(degp)


def _tc_combine_mm(p, b, w, n):
    npad, f = p.shape[1], w.shape[1]

    def body(p_ref, b_ref, w_ref, o_ref):
        h = jnp.maximum(p_ref[0, pl.ds(0, n), :] + p_ref[1, pl.ds(0, n), :]
                        + b_ref[...], 0.0)
        o_ref[pl.ds(0, n), :] = jnp.dot(h, w_ref[...],
                                        preferred_element_type=jnp.float32)
        o_ref[pl.ds(n, npad - n), :] = jnp.zeros((npad - n, f), jnp.float32)

    return pl.pallas_call(
        body, out_shape=jax.ShapeDtypeStruct((npad, f), jnp.float32))(
            p, b.reshape(1, -1), w)


def _tc_head(p, b3, wm1, bm1, wm2, bm2, n):
    def body(p_ref, b3_ref, wm1_ref, bm1_ref, wm2_ref, bm2_ref, o_ref):
        h3 = jnp.maximum(p_ref[0, pl.ds(0, n), :] + p_ref[1, pl.ds(0, n), :]
                         + b3_ref[...], 0.0)
        h4 = jnp.maximum(
            jnp.dot(h3, wm1_ref[...], preferred_element_type=jnp.float32)
            + bm1_ref[...], 0.0)
        h5 = (jnp.dot(h4, wm2_ref[...], preferred_element_type=jnp.float32)
              + bm2_ref[...])
        mn = jnp.min(h5)
        mx = jnp.max(h5)
        o_ref[...] = (h5 - mn) / (mx - mn)

    return pl.pallas_call(
        body, out_shape=jax.ShapeDtypeStruct((n, 1), jnp.float32))(
            p, b3.reshape(1, -1), wm1, bm1.reshape(1, -1), wm2,
            bm2.reshape(1, 1))


def kernel(x, edge_index, edge_probs, W1, b1, W2, b2, W3, b3, Wm1, bm1, Wm2, bm2):
    n = x.shape[0]
    hid = W1.shape[1]
    out_ch = W3.shape[1]
    i32 = jnp.int32

    loop = jnp.arange(n, dtype=i32)
    src = jnp.concatenate([edge_index[0].astype(i32), loop])
    dst = jnp.concatenate([edge_index[1].astype(i32), loop])
    ew = jnp.concatenate([edge_probs.astype(jnp.float32),
                          jnp.ones((n,), jnp.float32)])
    e = src.shape[0]

    real = _BLK - 1
    nblocks = ((-(-e // real) + 2 * _NW - 1) // (2 * _NW)) * (2 * _NW)
    pad = nblocks * real - e
    src = jnp.pad(src, (0, pad))
    dst = jnp.pad(dst, (0, pad), constant_values=n)
    ew = jnp.pad(ew, (0, pad))
    src = jnp.concatenate(
        [jnp.zeros((nblocks, 1), i32), src.reshape(nblocks, real)], 1).ravel()
    dst = jnp.concatenate(
        [jnp.full((nblocks, 1), n, i32), dst.reshape(nblocks, real)], 1).ravel()
    ew = jnp.concatenate(
        [jnp.zeros((nblocks, 1), jnp.float32), ew.reshape(nblocks, real)],
        1).ravel()

    npad = ((n + _NS * 8) // (_NS * 8)) * (_NS * 8)
    zeros_n = jnp.zeros((npad,), jnp.float32)
    zeros_nf = jnp.zeros((npad, hid), jnp.float32)

    srcb = src.reshape(nblocks, _BLK)
    dstb = dst.reshape(nblocks, _BLK)
    ewb = ew.reshape(nblocks, _BLK)
    recb = jnp.stack(
        [srcb, dstb, lax.bitcast_convert_type(ewb, i32)], axis=1)

    degp = _sc_deg(dstb, ewb, zeros_n)
    dis = _tc_dis(degp, n)
    norm = _sc_norm(recb, dis.reshape(npad))

    srcnorm = jnp.stack(
        [srcb, lax.bitcast_convert_type(norm.reshape(nblocks, _BLK), i32)],
        axis=1)

    u1 = _tc_mm(x, W1, npad)
    p1 = _sc_spmm(u1, srcnorm, dstb, zeros_nf)
    u2 = _tc_combine_mm(p1, b1, W2, n)
    p2 = _sc_spmm(u2, srcnorm, dstb, zeros_nf)
    w3p = jnp.concatenate(
        [W3, jnp.zeros((hid, hid - out_ch), jnp.float32)], axis=1)
    u3 = _tc_combine_mm(p2, b2, w3p, n)
    p3 = _sc_spmm(u3, srcnorm, dstb, zeros_nf)

    b3p = jnp.concatenate([b3, jnp.zeros((hid - out_ch,), jnp.float32)])
    wm1p = jnp.concatenate(
        [Wm1, jnp.zeros((hid - out_ch, Wm1.shape[1]), jnp.float32)], axis=0)
    return _tc_head(p3, b3p, wm1p, bm1, Wm2, bm2, n)
